# baseline (speedup 1.0000x reference)
import functools

import jax
import jax.numpy as jnp
from jax.experimental import pallas as pl
from jax.experimental.pallas import tpu as pltpu
from jax.experimental.pallas import tpu_sc as plsc

EMB = 32
N_FIELDS = 4
WIN = 1024


def kernel(x, T0, T1, T2, T3):
    F, B, S = x.shape
    x = x.astype(jnp.int32)

    xt = jnp.transpose(x, (0, 2, 1))

    mesh = plsc.VectorSubcoreMesh(
        core_axis_name="core", subcore_axis_name="subcore"
    )

    @functools.partial(
        pl.kernel,
        out_type=jax.ShapeDtypeStruct((B, S, N_FIELDS, EMB), jnp.float32),
        mesh=mesh,
        compiler_params=pltpu.CompilerParams(use_tc_tiling_on_sc=False),
    )
    def sc_gather(x_hbm, t0, t1, t2, t3, out_hbm):
        tabs = [t0, t1, t2, t3]
        for f in range(N_FIELDS):
            table = tabs[f]

            def body(i_vmem, o_vmem, table=table):
                pltpu.sync_copy(table.at[i_vmem.at[0, 0]], o_vmem.at[:, 0, 0])

            pltpu.emit_pipeline(
                body,
                grid=(S, B // WIN),
                in_specs=[
                    pl.BlockSpec(
                        (1, 1, WIN), index_map=lambda s, j, f=f: (f, s, j)
                    )
                ],
                out_specs=[
                    pl.BlockSpec(
                        (WIN, 1, 1, EMB),
                        index_map=lambda s, j, f=f: (j, s, f, 0),
                    )
                ],
                core_axis_name=("core", "subcore"),
                dimension_semantics=(pltpu.PARALLEL, pltpu.PARALLEL),
            )(x_hbm, out_hbm)

    out = sc_gather(xt, T0, T1, T2, T3)
    return out.reshape(B, S, N_FIELDS * EMB)


# speedup vs baseline: 4.9949x; 4.9949x over previous
import functools

import jax
import jax.numpy as jnp
from jax.experimental import pallas as pl
from jax.experimental.pallas import tpu as pltpu
from jax.experimental.pallas import tpu_sc as plsc

EMB = 32
N_FIELDS = 4
WIN = 1024


def kernel(x, T0, T1, T2, T3):
    F, B, S = x.shape
    x = x.astype(jnp.int32)

    xt = jnp.transpose(x, (0, 2, 1))

    mesh = plsc.VectorSubcoreMesh(
        core_axis_name="core", subcore_axis_name="subcore"
    )

    @functools.partial(
        pl.kernel,
        out_type=jax.ShapeDtypeStruct((B, S, N_FIELDS * EMB), jnp.float32),
        mesh=mesh,
        compiler_params=pltpu.CompilerParams(use_tc_tiling_on_sc=False),
    )
    def sc_gather(x_hbm, t0, t1, t2, t3, out_hbm):
        tabs = [t0, t1, t2, t3]
        for f in range(N_FIELDS):
            table = tabs[f]

            def body(i_vmem, o_vmem, table=table):
                pltpu.sync_copy(table.at[i_vmem.at[0, 0]], o_vmem.at[:, 0])

            pltpu.emit_pipeline(
                body,
                grid=(S, B // WIN),
                in_specs=[
                    pl.BlockSpec(
                        (1, 1, WIN), index_map=lambda s, j, f=f: (f, s, j)
                    )
                ],
                out_specs=[
                    pl.BlockSpec(
                        (WIN, 1, EMB),
                        index_map=lambda s, j, f=f: (j, s, f),
                    )
                ],
                core_axis_name=("core", "subcore"),
                dimension_semantics=(pltpu.PARALLEL, pltpu.PARALLEL),
            )(x_hbm, out_hbm)

    out = sc_gather(xt, T0, T1, T2, T3)
    return out
